# baseline (speedup 1.0000x reference)
import functools

import jax
import jax.numpy as jnp
from jax.experimental import pallas as pl
from jax.experimental.pallas import tpu as pltpu

_BM = 200
_OB = 4


def _gcn_body(x_ref, w_ref, b_ref, adj_ref, out_ref, xw_ref, obuf_ref,
              osem_ref):
    n = adj_ref.shape[1]
    nblk = n // _BM
    i = pl.program_id(0)
    slot = jax.lax.rem(i, _OB)

    def out_copy(k):
        return pltpu.make_async_copy(
            obuf_ref.at[jax.lax.rem(k, _OB)],
            out_ref.at[pl.ds(k * _BM, _BM), :],
            osem_ref.at[jax.lax.rem(k, _OB)],
        )

    @pl.when(i == 0)
    def _():
        xw_ref[...] = jnp.dot(
            x_ref[...], w_ref[...], preferred_element_type=jnp.float32
        )

    @pl.when(i >= _OB)
    def _():
        out_copy(i - _OB).wait()

    obuf_ref[slot] = (
        jnp.dot(adj_ref[...], xw_ref[...], preferred_element_type=jnp.float32)
        + b_ref[...]
    )
    out_copy(i).start()

    @pl.when(i == nblk - 1)
    def _():
        def drain(k, _):
            out_copy(nblk - _OB + k).wait()
            return _
        jax.lax.fori_loop(0, _OB, drain, None)


@functools.partial(jax.jit, static_argnames=())
def kernel(x, adj, w, b):
    n, f = x.shape
    h = w.shape[1]

    out = pl.pallas_call(
        _gcn_body,
        grid=(n // _BM,),
        in_specs=[
            pl.BlockSpec((n, f), lambda i: (0, 0)),
            pl.BlockSpec((f, h), lambda i: (0, 0)),
            pl.BlockSpec((1, h), lambda i: (0, 0)),
            pl.BlockSpec((_BM, n), lambda i: (i, 0)),
        ],
        out_specs=pl.BlockSpec(memory_space=pl.ANY),
        out_shape=jax.ShapeDtypeStruct((n, h), jnp.float32),
        scratch_shapes=[
            pltpu.VMEM((n, h), jnp.float32),
            pltpu.VMEM((_OB, _BM, h), jnp.float32),
            pltpu.SemaphoreType.DMA((_OB,)),
        ],
    )(x, w, b.reshape(1, h), adj)
    return out


# speedup vs baseline: 1.0038x; 1.0038x over previous
import functools

import jax
import jax.numpy as jnp
from jax.experimental import pallas as pl
from jax.experimental.pallas import tpu as pltpu

_BM = 400


def _gcn_body(x_ref, w_ref, b_ref, adj_ref, out_ref, xw_ref):
    @pl.when(pl.program_id(0) == 0)
    def _():
        xw_ref[...] = jnp.dot(
            x_ref[...], w_ref[...], preferred_element_type=jnp.float32
        )

    out_ref[...] = (
        jnp.dot(adj_ref[...], xw_ref[...], preferred_element_type=jnp.float32)
        + b_ref[...]
    )


@functools.partial(jax.jit, static_argnames=())
def kernel(x, adj, w, b):
    n, f = x.shape
    h = w.shape[1]

    out = pl.pallas_call(
        _gcn_body,
        grid=(n // _BM,),
        in_specs=[
            pl.BlockSpec((n, f), lambda i: (0, 0)),
            pl.BlockSpec((f, h), lambda i: (0, 0)),
            pl.BlockSpec((1, h), lambda i: (0, 0)),
            pl.BlockSpec((_BM, n), lambda i: (i, 0)),
        ],
        out_specs=pl.BlockSpec((_BM, h), lambda i: (i, 0)),
        out_shape=jax.ShapeDtypeStruct((n, h), jnp.float32),
        scratch_shapes=[pltpu.VMEM((n, h), jnp.float32)],
    )(x, w, b.reshape(1, h), adj)
    return out
